# baseline (speedup 1.0000x reference)
import functools

import jax
import jax.numpy as jnp
from jax import lax
from jax.experimental import pallas as pl
from jax.experimental.pallas import tpu as pltpu
from jax.experimental.pallas import tpu_sc as plsc

NUM_BINS = 128
NSLOT = 16
PROWS = NSLOT * NUM_BINS
_NC, _NS = 2, 16
_NW = _NC * _NS

_PI = 3.141592653589793
_HL = [(1.0, -1.0)] * 3 + [(2.0, 0.0)] + [(0.5, -0.5)] * 9 + [(_PI, 0.0)] * 3


def _proj_body(tn_ref, ta_ref, tv_ref, tg_ref, w_ref, p_ref):
    tabs = [tn_ref] * 3 + [ta_ref] + [tv_ref] * 9 + [tg_ref] * 3
    for s in range(NSLOT):
        t = tabs[s][...]
        w = w_ref[pl.ds(s * 128, 128), :]
        p_ref[pl.ds(s * NUM_BINS, NUM_BINS), :] = jnp.dot(
            t, w, preferred_element_type=jnp.float32).astype(jnp.bfloat16)


def _sc_gather_body(vert_hbm, fidx_hbm, out_hbm, vert_v, idx_v, out_v, *,
                    nf, nv):
    n3 = fidx_hbm.shape[0]
    n = n3 // 3
    chunk = n // _NW
    wid = lax.axis_index("s") * _NC + lax.axis_index("c")
    base = wid * chunk
    bofs3 = (base // nf) * nv * 3
    pltpu.sync_copy(vert_hbm, vert_v)
    pltpu.sync_copy(fidx_hbm.at[pl.ds(base * 3, chunk * 3)], idx_v)
    iota3 = lax.broadcasted_iota(jnp.int32, (16,), 0) * 3

    def body(j, carry):
        off = pl.multiple_of(j * 16, 16)
        for k in range(3):
            fk = plsc.load_gather(idx_v, [iota3 + (3 * off + k)])
            pos = fk * 3 + bofs3
            for c in range(3):
                out_v[pl.ds((k * 3 + c) * chunk + off, 16)] = plsc.load_gather(
                    vert_v, [pos + c])
        return carry

    lax.fori_loop(0, chunk // 16, body, 0)
    for r in range(9):
        pltpu.sync_copy(out_v.at[pl.ds(r * chunk, chunk)],
                        out_hbm.at[pl.ds(r * n + base, chunk)])


def _quant(feat, high, low):
    f = jnp.clip(feat, low, high)
    q = ((f - low) / (high - low) * NUM_BINS).astype(jnp.int32)
    return jnp.clip(q, 0, NUM_BINS - 1)


def _acos(x):
    r = 2.0 * lax.atan2(jnp.sqrt(1.0 - x * x), 1.0 + x)
    return jnp.where(x == -1.0, jnp.float32(_PI), r)


def _tc_main_body(coords_ref, p_ref, b_ref, o_ref, *, fb):
    eps = 1e-12
    c = [coords_ref[r:r + 1, :] for r in range(9)]

    def vtx(k):
        return c[3 * k], c[3 * k + 1], c[3 * k + 2]

    v0, v1, v2 = vtx(0), vtx(1), vtx(2)
    e1 = [v0[i] - v2[i] for i in range(3)]
    e2 = [v1[i] - v0[i] for i in range(3)]
    cx = e1[1] * e2[2] - e1[2] * e2[1]
    cy = e1[2] * e2[0] - e1[0] * e2[2]
    cz = e1[0] * e2[1] - e1[1] * e2[0]
    nrm = jnp.sqrt((cx * cx + cy * cy) + cz * cz)
    nsafe = jnp.maximum(nrm, eps)
    feats = [cx / nsafe, cy / nsafe, cz / nsafe, nrm * 0.5]
    feats += c
    verts = (v0, v1, v2)
    for k in range(3):
        a = [verts[(k + 1) % 3][i] - verts[k][i] for i in range(3)]
        b = [verts[(k + 2) % 3][i] - verts[k][i] for i in range(3)]
        na = jnp.maximum(jnp.sqrt((a[0] * a[0] + a[1] * a[1]) + a[2] * a[2]), eps)
        nb = jnp.maximum(jnp.sqrt((b[0] * b[0] + b[1] * b[1]) + b[2] * b[2]), eps)
        cos = ((a[0] / na) * (b[0] / nb) + (a[1] / na) * (b[1] / nb)) \
            + (a[2] / na) * (b[2] / nb)
        cos = jnp.clip(cos, -1.0, 1.0)
        feats.append(_acos(cos))

    qrows = [_quant(f, hi, lo) for f, (hi, lo) in zip(feats, _HL)]
    qmat = jnp.concatenate(qrows, axis=0).astype(jnp.bfloat16)
    ej = lax.broadcasted_iota(jnp.int32, (NSLOT, PROWS), 1)
    es = lax.broadcasted_iota(jnp.int32, (NSLOT, PROWS), 0)
    e_mat = ((ej >> 7) == es).astype(jnp.bfloat16)
    qb = lax.dot_general(qmat, e_mat, (((0,), (0,)), ((), ())),
                         preferred_element_type=jnp.float32)
    jmod = (lax.broadcasted_iota(jnp.int32, (fb, PROWS), 1)
            & (NUM_BINS - 1)).astype(jnp.float32)
    mht = (qb == jmod).astype(jnp.bfloat16)
    x = jnp.dot(mht, p_ref[...], preferred_element_type=jnp.float32)
    x = x + b_ref[0:1, :]
    xc = x * jnp.float32(0.7071067690849304)
    o_ref[...] = 0.5 * x * (1.0 + lax.erf(xc))


def kernel(vertices, faces, edges, face_masks, edge_masks,
           emb_norm, emb_area, emb_vertex, emb_angle, W_proj, b_proj):
    B, NV = vertices.shape[0], vertices.shape[1]
    NF = faces.shape[1]
    N = B * NF
    H = W_proj.shape[1]

    fidx_flat = jnp.where(face_masks[..., None], faces, 0).reshape(N * 3)
    vert_flat = vertices.reshape(B * NV * 3)
    b_pad = jnp.broadcast_to(b_proj[None, :], (8, H))

    p_tab = pl.pallas_call(
        _proj_body,
        out_shape=jax.ShapeDtypeStruct((PROWS, H), jnp.bfloat16),
    )(emb_norm[1:129], emb_area[1:129], emb_vertex[1:129], emb_angle[1:129],
      W_proj)

    coords_flat = pl.kernel(
        functools.partial(_sc_gather_body, nf=NF, nv=NV),
        out_type=jax.ShapeDtypeStruct((16 * N,), jnp.float32),
        mesh=plsc.VectorSubcoreMesh(core_axis_name="c", subcore_axis_name="s"),
        compiler_params=pltpu.CompilerParams(needs_layout_passes=False),
        scratch_types=[
            pltpu.VMEM((B * NV * 3,), jnp.float32),
            pltpu.VMEM((3 * (N // _NW),), jnp.int32),
            pltpu.VMEM((9 * (N // _NW),), jnp.float32),
        ],
    )(vert_flat, fidx_flat)
    coords = coords_flat.reshape(16, N)

    FB = 512
    out = pl.pallas_call(
        functools.partial(_tc_main_body, fb=FB),
        grid=(N // FB,),
        in_specs=[pl.BlockSpec((16, FB), lambda i: (0, i)),
                  pl.BlockSpec((PROWS, H), lambda i: (0, 0)),
                  pl.BlockSpec((8, H), lambda i: (0, 0))],
        out_specs=pl.BlockSpec((FB, H), lambda i: (i, 0)),
        out_shape=jax.ShapeDtypeStruct((N, H), jnp.float32),
    )(coords, p_tab, b_pad)
    return out.reshape(B, NF, H)


# speedup vs baseline: 1.2768x; 1.2768x over previous
import functools

import jax
import jax.numpy as jnp
from jax import lax
from jax.experimental import pallas as pl
from jax.experimental.pallas import tpu as pltpu
from jax.experimental.pallas import tpu_sc as plsc

NUM_BINS = 128
NSLOT = 16
PROWS = NSLOT * NUM_BINS
_NC, _NS = 2, 16
_NW = _NC * _NS

_PI = 3.141592653589793
_HL = [(1.0, -1.0)] * 3 + [(2.0, 0.0)] + [(0.5, -0.5)] * 9 + [(_PI, 0.0)] * 3


def _proj_body(tn_ref, ta_ref, tv_ref, tg_ref, w_ref, p_ref):
    tabs = [tn_ref] * 3 + [ta_ref] + [tv_ref] * 9 + [tg_ref] * 3
    for s in range(NSLOT):
        t = tabs[s][...]
        w = w_ref[pl.ds(s * 128, 128), :]
        p_ref[pl.ds(s * NUM_BINS, NUM_BINS), :] = jnp.dot(
            t, w, preferred_element_type=jnp.float32).astype(jnp.bfloat16)


def _sc_gather_body(vert_hbm, fidx_hbm, out_hbm, vert_v, idx_v, out_v, *,
                    nf, nv):
    n3 = fidx_hbm.shape[0]
    n = n3 // 3
    chunk = n // _NW
    wid = lax.axis_index("s") * _NC + lax.axis_index("c")
    base = wid * chunk
    bofs3 = (base // nf) * nv * 3
    pltpu.sync_copy(vert_hbm, vert_v)
    pltpu.sync_copy(fidx_hbm.at[pl.ds(base * 3, chunk * 3)], idx_v)
    iota3 = lax.broadcasted_iota(jnp.int32, (16,), 0) * 3

    def body(j, carry):
        off = pl.multiple_of(j * 16, 16)
        for k in range(3):
            fk = plsc.load_gather(idx_v, [iota3 + (3 * off + k)])
            pos = fk * 3 + bofs3
            for c in range(3):
                out_v[pl.ds((k * 3 + c) * chunk + off, 16)] = plsc.load_gather(
                    vert_v, [pos + c])
        return carry

    lax.fori_loop(0, chunk // 16, body, 0)
    for r in range(9):
        pltpu.sync_copy(out_v.at[pl.ds(r * chunk, chunk)],
                        out_hbm.at[pl.ds(r * n + base, chunk)])


def _quant(feat, high, low):
    f = jnp.clip(feat, low, high)
    q = ((f - low) / (high - low) * NUM_BINS).astype(jnp.int32)
    return jnp.clip(q, 0, NUM_BINS - 1)


def _acos(x):
    r = 2.0 * lax.atan2(jnp.sqrt(1.0 - x * x), 1.0 + x)
    return jnp.where(x == -1.0, jnp.float32(_PI), r)


def _tc_main_body(coords_ref, p_ref, b_ref, o_ref, *, fb):
    eps = 1e-12
    c = [coords_ref[r:r + 1, :] for r in range(9)]

    def vtx(k):
        return c[3 * k], c[3 * k + 1], c[3 * k + 2]

    v0, v1, v2 = vtx(0), vtx(1), vtx(2)
    e1 = [v0[i] - v2[i] for i in range(3)]
    e2 = [v1[i] - v0[i] for i in range(3)]
    cx = e1[1] * e2[2] - e1[2] * e2[1]
    cy = e1[2] * e2[0] - e1[0] * e2[2]
    cz = e1[0] * e2[1] - e1[1] * e2[0]
    nrm = jnp.sqrt((cx * cx + cy * cy) + cz * cz)
    nsafe = jnp.maximum(nrm, eps)
    feats = [cx / nsafe, cy / nsafe, cz / nsafe, nrm * 0.5]
    feats += c
    verts = (v0, v1, v2)
    for k in range(3):
        a = [verts[(k + 1) % 3][i] - verts[k][i] for i in range(3)]
        b = [verts[(k + 2) % 3][i] - verts[k][i] for i in range(3)]
        na = jnp.maximum(jnp.sqrt((a[0] * a[0] + a[1] * a[1]) + a[2] * a[2]), eps)
        nb = jnp.maximum(jnp.sqrt((b[0] * b[0] + b[1] * b[1]) + b[2] * b[2]), eps)
        cos = ((a[0] / na) * (b[0] / nb) + (a[1] / na) * (b[1] / nb)) \
            + (a[2] / na) * (b[2] / nb)
        cos = jnp.clip(cos, -1.0, 1.0)
        feats.append(_acos(cos))

    qrows = [_quant(f, hi, lo) for f, (hi, lo) in zip(feats, _HL)]
    iot = lax.broadcasted_iota(jnp.int32, (NUM_BINS, fb), 0)
    pieces = [(iot == q).astype(jnp.bfloat16) for q in qrows]
    mht = jnp.concatenate(pieces, axis=0)
    x = lax.dot_general(mht, p_ref[...], (((0,), (0,)), ((), ())),
                        preferred_element_type=jnp.float32)
    x = x + b_ref[0:1, :]
    xc = x * jnp.float32(0.7071067690849304)
    o_ref[...] = 0.5 * x * (1.0 + lax.erf(xc))


def kernel(vertices, faces, edges, face_masks, edge_masks,
           emb_norm, emb_area, emb_vertex, emb_angle, W_proj, b_proj):
    B, NV = vertices.shape[0], vertices.shape[1]
    NF = faces.shape[1]
    N = B * NF
    H = W_proj.shape[1]

    fidx_flat = (faces * face_masks[..., None]).reshape(N * 3)
    vert_flat = vertices.reshape(B * NV * 3)
    b_pad = jnp.broadcast_to(b_proj[None, :], (8, H))

    p_tab = pl.pallas_call(
        _proj_body,
        out_shape=jax.ShapeDtypeStruct((PROWS, H), jnp.bfloat16),
    )(emb_norm[1:129], emb_area[1:129], emb_vertex[1:129], emb_angle[1:129],
      W_proj)

    coords_flat = pl.kernel(
        functools.partial(_sc_gather_body, nf=NF, nv=NV),
        out_type=jax.ShapeDtypeStruct((16 * N,), jnp.float32),
        mesh=plsc.VectorSubcoreMesh(core_axis_name="c", subcore_axis_name="s"),
        compiler_params=pltpu.CompilerParams(needs_layout_passes=False),
        scratch_types=[
            pltpu.VMEM((B * NV * 3,), jnp.float32),
            pltpu.VMEM((3 * (N // _NW),), jnp.int32),
            pltpu.VMEM((9 * (N // _NW),), jnp.float32),
        ],
    )(vert_flat, fidx_flat)
    coords = coords_flat.reshape(16, N)

    FB = 1024
    out = pl.pallas_call(
        functools.partial(_tc_main_body, fb=FB),
        grid=(N // FB,),
        in_specs=[pl.BlockSpec((16, FB), lambda i: (0, i)),
                  pl.BlockSpec((PROWS, H), lambda i: (0, 0)),
                  pl.BlockSpec((8, H), lambda i: (0, 0))],
        out_specs=pl.BlockSpec((FB, H), lambda i: (i, 0)),
        out_shape=jax.ShapeDtypeStruct((N, H), jnp.float32),
    )(coords, p_tab, b_pad)
    return out.reshape(B, NF, H)


# speedup vs baseline: 1.2781x; 1.0010x over previous
import functools

import jax
import jax.numpy as jnp
from jax import lax
from jax.experimental import pallas as pl
from jax.experimental.pallas import tpu as pltpu
from jax.experimental.pallas import tpu_sc as plsc

NUM_BINS = 128
NSLOT = 16
PROWS = NSLOT * NUM_BINS
_NC, _NS = 2, 16
_NW = _NC * _NS

_PI = 3.141592653589793
_HL = [(1.0, -1.0)] * 3 + [(2.0, 0.0)] + [(0.5, -0.5)] * 9 + [(_PI, 0.0)] * 3


def _proj_body(tn_ref, ta_ref, tv_ref, tg_ref, w_ref, p_ref):
    tabs = [tn_ref] * 3 + [ta_ref] + [tv_ref] * 9 + [tg_ref] * 3
    for s in range(NSLOT):
        t = tabs[s][...]
        w = w_ref[pl.ds(s * 128, 128), :]
        p_ref[pl.ds(s * NUM_BINS, NUM_BINS), :] = jnp.dot(
            t, w, preferred_element_type=jnp.float32).astype(jnp.bfloat16)


def _sc_gather_body(vert_hbm, fidx_hbm, out_hbm, vert_v, idx_v, out_v, *,
                    nf, nv):
    n3 = fidx_hbm.shape[0]
    n = n3 // 3
    chunk = n // _NW
    wid = lax.axis_index("s") * _NC + lax.axis_index("c")
    base = wid * chunk
    bofs3 = (base // nf) * nv * 3
    pltpu.sync_copy(vert_hbm.at[pl.ds(bofs3, nv * 3)], vert_v)
    pltpu.sync_copy(fidx_hbm.at[pl.ds(base * 3, chunk * 3)], idx_v)
    iota3 = lax.broadcasted_iota(jnp.int32, (16,), 0) * 3

    def body(j, carry):
        off = pl.multiple_of(j * 16, 16)
        for k in range(3):
            fk = plsc.load_gather(idx_v, [iota3 + (3 * off + k)])
            pos = fk * 3
            for c in range(3):
                out_v[pl.ds((k * 3 + c) * chunk + off, 16)] = plsc.load_gather(
                    vert_v, [pos + c])
        return carry

    lax.fori_loop(0, chunk // 16, body, 0)
    for r in range(9):
        pltpu.sync_copy(out_v.at[pl.ds(r * chunk, chunk)],
                        out_hbm.at[pl.ds(r * n + base, chunk)])


def _quant(feat, high, low):
    f = jnp.clip(feat, low, high)
    q = ((f - low) / (high - low) * NUM_BINS).astype(jnp.int32)
    return jnp.clip(q, 0, NUM_BINS - 1)


def _acos(x):
    r = 2.0 * lax.atan2(jnp.sqrt(1.0 - x * x), 1.0 + x)
    return jnp.where(x == -1.0, jnp.float32(_PI), r)


def _tc_main_body(coords_ref, p_ref, b_ref, o_ref, *, fb):
    eps = 1e-12
    c = [coords_ref[r:r + 1, :] for r in range(9)]

    def vtx(k):
        return c[3 * k], c[3 * k + 1], c[3 * k + 2]

    v0, v1, v2 = vtx(0), vtx(1), vtx(2)
    e1 = [v0[i] - v2[i] for i in range(3)]
    e2 = [v1[i] - v0[i] for i in range(3)]
    cx = e1[1] * e2[2] - e1[2] * e2[1]
    cy = e1[2] * e2[0] - e1[0] * e2[2]
    cz = e1[0] * e2[1] - e1[1] * e2[0]
    nrm = jnp.sqrt((cx * cx + cy * cy) + cz * cz)
    nsafe = jnp.maximum(nrm, eps)
    feats = [cx / nsafe, cy / nsafe, cz / nsafe, nrm * 0.5]
    feats += c
    verts = (v0, v1, v2)
    for k in range(3):
        a = [verts[(k + 1) % 3][i] - verts[k][i] for i in range(3)]
        b = [verts[(k + 2) % 3][i] - verts[k][i] for i in range(3)]
        na = jnp.maximum(jnp.sqrt((a[0] * a[0] + a[1] * a[1]) + a[2] * a[2]), eps)
        nb = jnp.maximum(jnp.sqrt((b[0] * b[0] + b[1] * b[1]) + b[2] * b[2]), eps)
        cos = ((a[0] / na) * (b[0] / nb) + (a[1] / na) * (b[1] / nb)) \
            + (a[2] / na) * (b[2] / nb)
        cos = jnp.clip(cos, -1.0, 1.0)
        feats.append(_acos(cos))

    qrows = [_quant(f, hi, lo) for f, (hi, lo) in zip(feats, _HL)]
    iot = lax.broadcasted_iota(jnp.int32, (NUM_BINS, fb), 0)
    pieces = [(iot == q).astype(jnp.bfloat16) for q in qrows]
    mht = jnp.concatenate(pieces, axis=0)
    x = lax.dot_general(mht, p_ref[...], (((0,), (0,)), ((), ())),
                        preferred_element_type=jnp.float32)
    x = x + b_ref[0:1, :]
    xc = x * jnp.float32(0.7071067690849304)
    o_ref[...] = 0.5 * x * (1.0 + lax.erf(xc))


def kernel(vertices, faces, edges, face_masks, edge_masks,
           emb_norm, emb_area, emb_vertex, emb_angle, W_proj, b_proj):
    B, NV = vertices.shape[0], vertices.shape[1]
    NF = faces.shape[1]
    N = B * NF
    H = W_proj.shape[1]

    fidx_flat = (faces * face_masks[..., None]).reshape(N * 3)
    vert_flat = vertices.reshape(B * NV * 3)
    b_pad = jnp.broadcast_to(b_proj[None, :], (8, H))

    p_tab = pl.pallas_call(
        _proj_body,
        out_shape=jax.ShapeDtypeStruct((PROWS, H), jnp.bfloat16),
    )(emb_norm[1:129], emb_area[1:129], emb_vertex[1:129], emb_angle[1:129],
      W_proj)

    coords_flat = pl.kernel(
        functools.partial(_sc_gather_body, nf=NF, nv=NV),
        out_type=jax.ShapeDtypeStruct((16 * N,), jnp.float32),
        mesh=plsc.VectorSubcoreMesh(core_axis_name="c", subcore_axis_name="s"),
        compiler_params=pltpu.CompilerParams(needs_layout_passes=False),
        scratch_types=[
            pltpu.VMEM((NV * 3,), jnp.float32),
            pltpu.VMEM((3 * (N // _NW),), jnp.int32),
            pltpu.VMEM((9 * (N // _NW),), jnp.float32),
        ],
    )(vert_flat, fidx_flat)
    coords = coords_flat.reshape(16, N)

    FB = 512
    out = pl.pallas_call(
        functools.partial(_tc_main_body, fb=FB),
        grid=(N // FB,),
        in_specs=[pl.BlockSpec((16, FB), lambda i: (0, i)),
                  pl.BlockSpec((PROWS, H), lambda i: (0, 0)),
                  pl.BlockSpec((8, H), lambda i: (0, 0))],
        out_specs=pl.BlockSpec((FB, H), lambda i: (i, 0)),
        out_shape=jax.ShapeDtypeStruct((N, H), jnp.float32),
    )(coords, p_tab, b_pad)
    return out.reshape(B, NF, H)
